# Initial kernel scaffold; baseline (speedup 1.0000x reference)
import jax
import jax.numpy as jnp
from jax.experimental import pallas as pl


def kernel(boxes, scores):
    raise NotImplementedError("write your pallas kernel here")


# speedup vs baseline: 189.3394x; 189.3394x over previous
import functools

import jax
import jax.numpy as jnp
from jax.experimental import pallas as pl
from jax.experimental.pallas import tpu as pltpu

_N = 5000
_NP = 5120
_T = 1024
_NT = _NP // _T
_K = 1000
_KP = 1024
_TH = 0.7


def _iou_gt(rc, cc):
    x1a, y1a, x2a, y2a = rc[:, 0:1], rc[:, 1:2], rc[:, 2:3], rc[:, 3:4]
    x1b, y1b, x2b, y2b = cc[0:1, :], cc[1:2, :], cc[2:3, :], cc[3:4, :]
    area_a = (x2a - x1a) * (y2a - y1a)
    area_b = (x2b - x1b) * (y2b - y1b)
    xx1 = jnp.maximum(x1a, x1b)
    yy1 = jnp.maximum(y1a, y1b)
    xx2 = jnp.minimum(x2a, x2b)
    yy2 = jnp.minimum(y2a, y2b)
    w = jnp.clip(xx2 - xx1, 0.0)
    h = jnp.clip(yy2 - yy1, 0.0)
    inter = w * h
    iou = inter / (area_a + area_b - inter + 1e-9)
    return iou > _TH


def _matvec(k, m):
    return jax.lax.dot_general(
        k, m, (((1,), (0,)), ((), ())),
        preferred_element_type=jnp.float32)


def _nms_body(rows_ref, cols_ref, out_ref, supp_ref):
    supp_ref[:, :] = jnp.zeros((1, _NP), jnp.float32)
    out_ref[0] = jnp.zeros((_KP, 8), jnp.float32)

    i0 = jax.lax.broadcasted_iota(jnp.int32, (_T, _T), 0)
    i1 = jax.lax.broadcasted_iota(jnp.int32, (_T, _T), 1)
    tri_strict = i0 < i1
    tri_incl_f = (i0 <= i1).astype(jnp.float32)

    def tile_step(t, offset):
        off = t * _T
        cc_t = rows_ref[0, :, pl.ds(off, _T)]
        rc_t = cols_ref[0, pl.ds(off, _T), :]
        col_ids = jax.lax.broadcasted_iota(jnp.int32, (1, _T), 1) + off
        active = (supp_ref[:, pl.ds(off, _T)] == 0.0) & (col_ids < _N)
        active_f = active.astype(jnp.float32)

        over = _iou_gt(rc_t, cc_t) & tri_strict
        over_f = over.astype(jnp.float32)

        def fix_cond(s):
            return s[1]

        def fix_body(s):
            k, _ = s
            sup = _matvec(k, over_f)
            k_new = jnp.where(sup > 0.5, 0.0, active_f)
            return k_new, jnp.any(k_new != k)

        k, _ = jax.lax.while_loop(
            fix_cond, fix_body, (active_f, jnp.array(True)))

        csum = _matvec(k, tri_incl_f)
        rank = csum - 1.0 + offset
        rank = jnp.where((k > 0.5) & (rank < float(_K)), rank, -1.0)
        rank_i = rank.astype(jnp.int32)
        onehot = (jax.lax.broadcasted_iota(jnp.int32, (_KP, _T), 0)
                  == rank_i).astype(jnp.float32)
        out_ref[0] += jax.lax.dot_general(
            onehot, rc_t, (((1,), (0,)), ((), ())),
            precision=jax.lax.Precision.HIGHEST,
            preferred_element_type=jnp.float32)

        def cross(s, _):
            offs = s * _T
            cc_s = rows_ref[0, :, pl.ds(offs, _T)]
            over2 = _iou_gt(rc_t, cc_s).astype(jnp.float32)
            sup = _matvec(k, over2)
            cur = supp_ref[:, pl.ds(offs, _T)]
            supp_ref[:, pl.ds(offs, _T)] = jnp.where(sup > 0.5, 1.0, cur)
            return 0

        jax.lax.fori_loop(t + 1, _NT, cross, 0)
        return offset + jnp.sum(k)

    jax.lax.fori_loop(0, _NT, tile_step, jnp.float32(0.0))


@functools.partial(jax.jit, static_argnames=("interpret",))
def _run(boxes, scores, interpret=False):
    b = boxes.shape[0]
    order = jnp.argsort(-scores, axis=-1)
    sb = jnp.take_along_axis(boxes, order[..., None], axis=1)
    sb = jnp.pad(sb, ((0, 0), (0, _NP - _N), (0, 0)))
    cols = jnp.pad(sb, ((0, 0), (0, 0), (0, 4)))
    rows = jnp.pad(jnp.swapaxes(sb, 1, 2), ((0, 0), (0, 4), (0, 0)))

    res = pl.pallas_call(
        _nms_body,
        grid=(b,),
        in_specs=[
            pl.BlockSpec((1, 8, _NP), lambda i: (i, 0, 0)),
            pl.BlockSpec((1, _NP, 8), lambda i: (i, 0, 0)),
        ],
        out_specs=pl.BlockSpec((1, _KP, 8), lambda i: (i, 0, 0)),
        out_shape=jax.ShapeDtypeStruct((b, _KP, 8), jnp.float32),
        scratch_shapes=[pltpu.VMEM((1, _NP), jnp.float32)],
        interpret=interpret,
    )(rows, cols)

    kept = res[:, :_K, :4]
    batch_idx = jnp.broadcast_to(
        jnp.arange(b, dtype=boxes.dtype)[:, None, None], (b, _K, 1))
    return jnp.concatenate([batch_idx, kept], axis=-1)


def kernel(boxes, scores):
    return _run(boxes, scores)


# speedup vs baseline: 212.8285x; 1.1241x over previous
import functools

import jax
import jax.numpy as jnp
from jax.experimental import pallas as pl
from jax.experimental.pallas import tpu as pltpu

_N = 5000
_NP = 5120
_T = 1024
_NT = _NP // _T
_K = 1000
_KP = 1024
_TH = 0.7


def _iou_gt(rc, cc):
    x1a, y1a, x2a, y2a = rc[:, 0:1], rc[:, 1:2], rc[:, 2:3], rc[:, 3:4]
    x1b, y1b, x2b, y2b = cc[0:1, :], cc[1:2, :], cc[2:3, :], cc[3:4, :]
    area_a = (x2a - x1a) * (y2a - y1a)
    area_b = (x2b - x1b) * (y2b - y1b)
    xx1 = jnp.maximum(x1a, x1b)
    yy1 = jnp.maximum(y1a, y1b)
    xx2 = jnp.minimum(x2a, x2b)
    yy2 = jnp.minimum(y2a, y2b)
    w = jnp.clip(xx2 - xx1, 0.0)
    h = jnp.clip(yy2 - yy1, 0.0)
    inter = w * h
    iou = inter / (area_a + area_b - inter + 1e-9)
    return iou > _TH


def _matvec(k, m):
    return jax.lax.dot_general(
        k, m, (((1,), (0,)), ((), ())),
        preferred_element_type=jnp.float32)


def _nms_body(rows_ref, cols_ref, out_ref, supp_ref):
    supp_ref[:, :] = jnp.zeros((1, _NP), jnp.float32)
    out_ref[0] = jnp.zeros((_KP, 8), jnp.float32)

    i0 = jax.lax.broadcasted_iota(jnp.int32, (_T, _T), 0)
    i1 = jax.lax.broadcasted_iota(jnp.int32, (_T, _T), 1)
    tri_strict = i0 < i1
    tri_incl_f = (i0 <= i1).astype(jnp.float32)

    def tile_step(t, offset):
        off = t * _T
        cc_t = rows_ref[0, :, pl.ds(off, _T)]
        rc_t = cols_ref[0, pl.ds(off, _T), :]
        col_ids = jax.lax.broadcasted_iota(jnp.int32, (1, _T), 1) + off
        active = (supp_ref[:, pl.ds(off, _T)] == 0.0) & (col_ids < _N)
        active_f = active.astype(jnp.float32)

        over = _iou_gt(rc_t, cc_t) & tri_strict
        over_f = over.astype(jnp.float32)

        def fix_cond(s):
            return s[1]

        def fix_body(s):
            k, _ = s
            sup = _matvec(k, over_f)
            k_new = jnp.where(sup > 0.5, 0.0, active_f)
            return k_new, jnp.any(k_new != k)

        k, _ = jax.lax.while_loop(
            fix_cond, fix_body, (active_f, jnp.array(True)))

        csum = _matvec(k, tri_incl_f)
        rank = csum - 1.0 + offset
        rank = jnp.where((k > 0.5) & (rank < float(_K)), rank, -1.0)
        rank_i = rank.astype(jnp.int32)
        onehot = (jax.lax.broadcasted_iota(jnp.int32, (_KP, _T), 0)
                  == rank_i).astype(jnp.bfloat16)
        rc_hi = rc_t.astype(jnp.bfloat16)
        rc_mid = (rc_t - rc_hi.astype(jnp.float32)).astype(jnp.bfloat16)
        dn = (((1,), (0,)), ((), ()))
        out_ref[0] += (
            jax.lax.dot_general(onehot, rc_hi, dn,
                                preferred_element_type=jnp.float32)
            + jax.lax.dot_general(onehot, rc_mid, dn,
                                  preferred_element_type=jnp.float32))

        def cross(s, _):
            offs = s * _T
            cc_s = rows_ref[0, :, pl.ds(offs, _T)]
            over2 = _iou_gt(rc_t, cc_s).astype(jnp.float32)
            sup = _matvec(k, over2)
            cur = supp_ref[:, pl.ds(offs, _T)]
            supp_ref[:, pl.ds(offs, _T)] = jnp.where(sup > 0.5, 1.0, cur)
            return 0

        jax.lax.fori_loop(t + 1, _NT, cross, 0)
        return offset + jnp.sum(k)

    jax.lax.fori_loop(0, _NT, tile_step, jnp.float32(0.0))


@functools.partial(jax.jit, static_argnames=("interpret",))
def _run(boxes, scores, interpret=False):
    b = boxes.shape[0]
    order = jnp.argsort(-scores, axis=-1)
    sb = jnp.take_along_axis(boxes, order[..., None], axis=1)
    sb = jnp.pad(sb, ((0, 0), (0, _NP - _N), (0, 0)))
    cols = jnp.pad(sb, ((0, 0), (0, 0), (0, 4)))
    rows = jnp.pad(jnp.swapaxes(sb, 1, 2), ((0, 0), (0, 4), (0, 0)))

    res = pl.pallas_call(
        _nms_body,
        grid=(b,),
        in_specs=[
            pl.BlockSpec((1, 8, _NP), lambda i: (i, 0, 0)),
            pl.BlockSpec((1, _NP, 8), lambda i: (i, 0, 0)),
        ],
        out_specs=pl.BlockSpec((1, _KP, 8), lambda i: (i, 0, 0)),
        out_shape=jax.ShapeDtypeStruct((b, _KP, 8), jnp.float32),
        scratch_shapes=[pltpu.VMEM((1, _NP), jnp.float32)],
        interpret=interpret,
    )(rows, cols)

    kept = res[:, :_K, :4]
    batch_idx = jnp.broadcast_to(
        jnp.arange(b, dtype=boxes.dtype)[:, None, None], (b, _K, 1))
    return jnp.concatenate([batch_idx, kept], axis=-1)


def kernel(boxes, scores):
    return _run(boxes, scores)


# speedup vs baseline: 387.6637x; 1.8215x over previous
import functools

import jax
import jax.numpy as jnp
from jax.experimental import pallas as pl
from jax.experimental.pallas import tpu as pltpu

_N = 5000
_NP = 5120
_T = 1024
_NT = _NP // _T
_K = 1000
_KP = 1024
_TH = 0.7


def _iou_gt(rc, cc):
    x1a, y1a, x2a, y2a = rc[:, 0:1], rc[:, 1:2], rc[:, 2:3], rc[:, 3:4]
    x1b, y1b, x2b, y2b = cc[0:1, :], cc[1:2, :], cc[2:3, :], cc[3:4, :]
    area_a = (x2a - x1a) * (y2a - y1a)
    area_b = (x2b - x1b) * (y2b - y1b)
    xx1 = jnp.maximum(x1a, x1b)
    yy1 = jnp.maximum(y1a, y1b)
    xx2 = jnp.minimum(x2a, x2b)
    yy2 = jnp.minimum(y2a, y2b)
    w = jnp.clip(xx2 - xx1, 0.0)
    h = jnp.clip(yy2 - yy1, 0.0)
    inter = w * h
    iou = inter / (area_a + area_b - inter + 1e-9)
    return iou > _TH


def _matvec(k, m):
    return jax.lax.dot_general(
        k, m, (((1,), (0,)), ((), ())),
        preferred_element_type=jnp.float32)


def _nms_body(rows_ref, cols_ref, out_ref, keep_ref):
    out_ref[0] = jnp.zeros((_KP, 8), jnp.float32)

    i0 = jax.lax.broadcasted_iota(jnp.int32, (_T, _T), 0)
    i1 = jax.lax.broadcasted_iota(jnp.int32, (_T, _T), 1)
    tri_strict = i0 < i1
    tri_incl = (i0 <= i1).astype(jnp.bfloat16)

    def tile_step(t, offset):
        def work(offset):
            off = t * _T
            cc_t = rows_ref[0, :, pl.ds(off, _T)]
            rc_t = cols_ref[0, pl.ds(off, _T), :]
            col_ids = jax.lax.broadcasted_iota(jnp.int32, (1, _T), 1) + off

            def pull(u, sup):
                offu = u * _T
                rc_u = cols_ref[0, pl.ds(offu, _T), :]
                k_u = keep_ref[:, pl.ds(offu, _T)].astype(jnp.bfloat16)
                over2 = _iou_gt(rc_u, cc_t).astype(jnp.bfloat16)
                return sup + _matvec(k_u, over2)

            sup0 = jax.lax.fori_loop(
                0, t, pull, jnp.zeros((1, _T), jnp.float32))
            active_f = jnp.where(
                (sup0 < 0.5) & (col_ids < _N), 1.0, 0.0)

            over_f = (_iou_gt(rc_t, cc_t) & tri_strict).astype(jnp.bfloat16)

            def fix_cond(s):
                return s[1]

            def fix_body(s):
                k, _ = s
                sup = _matvec(k.astype(jnp.bfloat16), over_f)
                k_new = jnp.where(sup > 0.5, 0.0, active_f)
                return k_new, jnp.any(k_new != k)

            k, _ = jax.lax.while_loop(
                fix_cond, fix_body, (active_f, jnp.array(True)))
            keep_ref[:, pl.ds(off, _T)] = k

            k_bf = k.astype(jnp.bfloat16)
            csum = _matvec(k_bf, tri_incl)
            rank = csum - 1.0 + offset
            rank = jnp.where((k > 0.5) & (rank < float(_K)), rank, -1.0)
            rank_i = rank.astype(jnp.int32)
            onehot = (jax.lax.broadcasted_iota(jnp.int32, (_KP, _T), 0)
                      == rank_i).astype(jnp.bfloat16)
            rc_hi = rc_t.astype(jnp.bfloat16)
            rc_mid = (rc_t - rc_hi.astype(jnp.float32)).astype(jnp.bfloat16)
            dn = (((1,), (0,)), ((), ()))
            out_ref[0] += (
                jax.lax.dot_general(onehot, rc_hi, dn,
                                    preferred_element_type=jnp.float32)
                + jax.lax.dot_general(onehot, rc_mid, dn,
                                      preferred_element_type=jnp.float32))
            return offset + jnp.sum(k)

        return jax.lax.cond(offset < float(_K), work, lambda o: o, offset)

    jax.lax.fori_loop(0, _NT, tile_step, jnp.float32(0.0))


@functools.partial(jax.jit, static_argnames=("interpret",))
def _run(boxes, scores, interpret=False):
    b = boxes.shape[0]
    order = jnp.argsort(-scores, axis=-1)
    sb = jnp.take_along_axis(boxes, order[..., None], axis=1)
    sb = jnp.pad(sb, ((0, 0), (0, _NP - _N), (0, 0)))
    cols = jnp.pad(sb, ((0, 0), (0, 0), (0, 4)))
    rows = jnp.pad(jnp.swapaxes(sb, 1, 2), ((0, 0), (0, 4), (0, 0)))

    res = pl.pallas_call(
        _nms_body,
        grid=(b,),
        in_specs=[
            pl.BlockSpec((1, 8, _NP), lambda i: (i, 0, 0)),
            pl.BlockSpec((1, _NP, 8), lambda i: (i, 0, 0)),
        ],
        out_specs=pl.BlockSpec((1, _KP, 8), lambda i: (i, 0, 0)),
        out_shape=jax.ShapeDtypeStruct((b, _KP, 8), jnp.float32),
        scratch_shapes=[pltpu.VMEM((1, _NP), jnp.float32)],
        interpret=interpret,
    )(rows, cols)

    kept = res[:, :_K, :4]
    batch_idx = jnp.broadcast_to(
        jnp.arange(b, dtype=boxes.dtype)[:, None, None], (b, _K, 1))
    return jnp.concatenate([batch_idx, kept], axis=-1)


def kernel(boxes, scores):
    return _run(boxes, scores)
